# baseline (speedup 1.0000x reference)
import functools

import jax
import jax.numpy as jnp
import numpy as np
from jax import lax
from jax.experimental import pallas as pl
from jax.experimental.pallas import tpu as pltpu
from jax.experimental.pallas import tpu_sc as plsc

B = 16384
N_VERTS = 778
N_ANCHORS = 32
ROW_W = N_VERTS * 3
OUT_W = N_ANCHORS * 3
LANES = 16
NUM_CORES = 2
NUM_SUBCORES = 16
NUM_WORKERS = NUM_CORES * NUM_SUBCORES
ROWS_PER_WORKER = B // NUM_WORKERS
CHUNK = 16
NCHUNK = ROWS_PER_WORKER // CHUNK
N_OVEC = OUT_W // LANES
N_VEC = 3 * N_OVEC

_MESH = plsc.VectorSubcoreMesh(
    core_axis_name="c", subcore_axis_name="s",
    num_cores=NUM_CORES, num_subcores=NUM_SUBCORES)


@functools.partial(
    pl.kernel,
    out_type=jax.ShapeDtypeStruct((B, OUT_W), jnp.float32),
    mesh=_MESH,
    scratch_types=[
        pltpu.VMEM((2, CHUNK, ROW_W), jnp.float32),
        pltpu.VMEM((2, CHUNK, OUT_W), jnp.float32),
        pltpu.VMEM((N_VEC, LANES), jnp.int32),
        pltpu.VMEM((N_VEC, LANES), jnp.float32),
        pltpu.SemaphoreType.DMA,
        pltpu.SemaphoreType.DMA,
        pltpu.SemaphoreType.DMA,
        pltpu.SemaphoreType.DMA,
    ],
    compiler_params=pltpu.CompilerParams(use_tc_tiling_on_sc=False,
                                         needs_layout_passes=False),
)
def _anchor_sc(verts_hbm, gidx_hbm, wv_hbm, out_hbm, buf, obuf, gidx_v, wv_v,
               isem0, isem1, osem0, osem1):
    wid = lax.axis_index("s") * NUM_CORES + lax.axis_index("c")
    base = wid * ROWS_PER_WORKER

    pltpu.sync_copy(gidx_hbm, gidx_v)
    pltpu.sync_copy(wv_hbm, wv_v)

    gidx = [gidx_v[j, :] for j in range(N_VEC)]
    wvec = [wv_v[j, :] for j in range(N_VEC)]

    isems = [isem0, isem1]
    osems = [osem0, osem1]

    def start_in(g, slot):
        return pltpu.async_copy(
            verts_hbm.at[pl.ds(base + g * CHUNK, CHUNK)],
            buf.at[slot], isems[slot])

    def compute_chunk(slot):
        bslot = buf.at[slot]
        oslot = obuf.at[slot]

        def row_body(r, carry):
            r_splat = lax.broadcast(r, (LANES,))
            for o in range(N_OVEC):
                acc = (plsc.load_gather(bslot, [r_splat, gidx[3 * o]])
                       * wvec[3 * o])
                acc += (plsc.load_gather(bslot, [r_splat, gidx[3 * o + 1]])
                        * wvec[3 * o + 1])
                acc += (plsc.load_gather(bslot, [r_splat, gidx[3 * o + 2]])
                        * wvec[3 * o + 2])
                oslot[r, pl.ds(o * LANES, LANES)] = acc
            return carry

        lax.fori_loop(0, CHUNK, row_body, 0)

    in_flight = {}
    out_flight = {}
    in_flight[0] = start_in(0, 0)
    for g in range(NCHUNK):
        slot = g % 2
        if g + 1 < NCHUNK:
            in_flight[(g + 1) % 2] = start_in(g + 1, (g + 1) % 2)
        in_flight[slot].wait()
        if g >= 2:
            out_flight[slot].wait()
        compute_chunk(slot)
        out_flight[slot] = pltpu.async_copy(
            obuf.at[slot],
            out_hbm.at[pl.ds(base + g * CHUNK, CHUNK)], osems[slot])
    out_flight[0].wait()
    out_flight[1].wait()


def kernel(vertices, face_vert_idx, anchor_weight):
    verts2d = vertices.reshape(B, ROW_W)
    fvi_flat = face_vert_idx.reshape(OUT_W).astype(jnp.int32)
    w_flat = anchor_weight.reshape(OUT_W).astype(jnp.float32)

    p = np.arange(OUT_W)
    pa, pc = p // 3, p % 3
    gidx_rows, wv_rows = [], []
    for o in range(N_OVEC):
        sl = slice(o * LANES, (o + 1) * LANES)
        for k in range(3):
            sel = jnp.asarray(pa[sl] * 3 + k, dtype=jnp.int32)
            gidx_rows.append(jnp.take(fvi_flat, sel) * 3
                             + jnp.asarray(pc[sl], dtype=jnp.int32))
            wv_rows.append(jnp.take(w_flat, sel))
    gidx_all = jnp.stack(gidx_rows)
    wv_all = jnp.stack(wv_rows)

    out = _anchor_sc(verts2d, gidx_all, wv_all)
    return out.reshape(B, N_ANCHORS, 3)


# speedup vs baseline: 28.1235x; 28.1235x over previous
import functools

import jax
import jax.numpy as jnp
from jax import lax
from jax.experimental import pallas as pl
from jax.experimental.pallas import tpu as pltpu
from jax.experimental.pallas import tpu_sc as plsc

B = 16384
N_VERTS = 778
N_ANCHORS = 32
LANES = 16
NUM_CORES = 2
NUM_SUBCORES = 16
NUM_WORKERS = NUM_CORES * NUM_SUBCORES
BW = B // NUM_WORKERS
NJ = BW // LANES

_MESH = plsc.VectorSubcoreMesh(
    core_axis_name="c", subcore_axis_name="s",
    num_cores=NUM_CORES, num_subcores=NUM_SUBCORES)


@functools.partial(
    pl.kernel,
    out_type=jax.ShapeDtypeStruct((3, N_ANCHORS, B), jnp.float32),
    mesh=_MESH,
    scratch_types=[
        pltpu.VMEM((2, 3, 3, BW), jnp.float32),
        pltpu.VMEM((2, 3, BW), jnp.float32),
        pltpu.VMEM((96,), jnp.int32),
        pltpu.VMEM((96,), jnp.float32),
        pltpu.SemaphoreType.DMA,
        pltpu.SemaphoreType.DMA,
        pltpu.SemaphoreType.DMA,
        pltpu.SemaphoreType.DMA,
    ],
    compiler_params=pltpu.CompilerParams(use_tc_tiling_on_sc=True,
                                         needs_layout_passes=False),
)
def _anchor_sc(verts_hbm, fvi_hbm, w_hbm, out_hbm, gbuf, obuf, fvi_v, w_v,
               isem0, isem1, osem0, osem1):
    wid = lax.axis_index("s") * NUM_CORES + lax.axis_index("c")
    col0 = wid * BW

    pltpu.sync_copy(fvi_hbm, fvi_v)
    pltpu.sync_copy(w_hbm, w_v)

    lane_iota = lax.iota(jnp.int32, LANES)
    isems = [isem0, isem1]
    osems = [osem0, osem1]

    def vert_index(a, k):
        j = 3 * a + k
        grp = fvi_v[pl.ds((j // LANES) * LANES, LANES)]
        masked = jnp.where(lane_iota == (j % LANES), grp, 0)
        return jnp.sum(masked, axis=0)

    def weight_splat(a, k):
        j = 3 * a + k
        grp = w_v[pl.ds((j // LANES) * LANES, LANES)]
        lane = jnp.full((LANES, 1), j % LANES, jnp.int32)
        return lax.gather(
            grp, lane,
            lax.GatherDimensionNumbers(offset_dims=(),
                                       collapsed_slice_dims=(0,),
                                       start_index_map=(0,)),
            slice_sizes=(1,),
            mode=lax.GatherScatterMode.PROMISE_IN_BOUNDS)

    def start_in(a, slot):
        copies = []
        for k in range(3):
            v = vert_index(a, k)
            copies.append(pltpu.async_copy(
                verts_hbm.at[:, v, pl.ds(col0, BW)],
                gbuf.at[slot, k], isems[slot]))
        return copies

    def compute(a, slot):
        w0 = weight_splat(a, 0)
        w1 = weight_splat(a, 1)
        w2 = weight_splat(a, 2)
        g = gbuf.at[slot]
        o = obuf.at[slot]

        def jbody(j, carry):
            off = j * LANES
            for c in range(3):
                acc = g[0, c, pl.ds(off, LANES)] * w0
                acc += g[1, c, pl.ds(off, LANES)] * w1
                acc += g[2, c, pl.ds(off, LANES)] * w2
                o[c, pl.ds(off, LANES)] = acc
            return carry

        lax.fori_loop(0, NJ, jbody, 0)

    in_flight = {}
    out_flight = {}
    in_flight[0] = start_in(0, 0)
    for a in range(N_ANCHORS):
        slot = a % 2
        if a + 1 < N_ANCHORS:
            in_flight[(a + 1) % 2] = start_in(a + 1, (a + 1) % 2)
        for cp in in_flight[slot]:
            cp.wait()
        if a >= 2:
            out_flight[slot].wait()
        compute(a, slot)
        out_flight[slot] = pltpu.async_copy(
            obuf.at[slot],
            out_hbm.at[:, a, pl.ds(col0, BW)], osems[slot])
    out_flight[0].wait()
    out_flight[1].wait()


def kernel(vertices, face_vert_idx, anchor_weight):
    verts_t = jnp.transpose(vertices, (2, 1, 0))
    fvi_flat = face_vert_idx.reshape(96).astype(jnp.int32)
    w_flat = anchor_weight.reshape(96).astype(jnp.float32)
    out_t = _anchor_sc(verts_t, fvi_flat, w_flat)
    return jnp.transpose(out_t, (2, 1, 0))
